# baseline (speedup 1.0000x reference)
import functools

import jax
import jax.numpy as jnp
from jax import lax
from jax.experimental import pallas as pl
from jax.experimental.pallas import tpu as pltpu
from jax.experimental.pallas import tpu_sc as plsc

DIM = 64
_S0, _S1 = 4096, 200

_info = plsc.get_sparse_core_info()
_NC, _NS = _info.num_cores, _info.num_subcores
_NW = _NC * _NS
_TPW = _S0 // _NW

_mesh = plsc.VectorSubcoreMesh(core_axis_name="c", subcore_axis_name="s")


@functools.partial(
    pl.kernel,
    mesh=_mesh,
    out_type=jax.ShapeDtypeStruct((_S1, 8, _S0 // 128, 8, 128), jnp.float32),
    scratch_types=[
        pltpu.VMEM((2, 128), jnp.int32),
        pltpu.VMEM((2, 128, 128), jnp.float32),
        pltpu.VMEM((2, 8, 1, 8, 128), jnp.float32),
        pltpu.SemaphoreType.DMA,
        pltpu.SemaphoreType.DMA,
    ],
    compiler_params=pltpu.CompilerParams(needs_layout_passes=False),
)
def _emb_lookup(sent_hbm, table_hbm, out_hbm, idx_v, rows_v, trans_v, sem0, sem1):
    wid = lax.axis_index("s") * _NC + lax.axis_index("c")
    col0 = pl.multiple_of(wid * _TPW, _TPW)
    sems = (sem0, sem1)
    lanes = [jnp.arange(16, dtype=jnp.int32) + 16 * j for j in range(8)]

    def fire(s1, b):
        pltpu.sync_copy(sent_hbm.at[s1, pl.ds(col0, _TPW)], idx_v.at[b])
        pltpu.async_copy(table_hbm.at[idx_v.at[b]], rows_v.at[b], sems[b])

    def proc(s1, b):
        pltpu.make_async_copy(
            table_hbm.at[idx_v.at[b]], rows_v.at[b], sems[b]
        ).wait()

        def trow(fd, carry):
            for fm in range(8):
                f = fd * 8 + fm
                col = jnp.full((16,), f, dtype=jnp.int32)
                for j in range(8):
                    v = plsc.load_gather(rows_v.at[b], [lanes[j], col])
                    trans_v[b, fd, 0, fm, pl.ds(16 * j, 16)] = v
            return carry

        lax.fori_loop(0, 8, trow, 0)
        pltpu.sync_copy(
            trans_v.at[b],
            out_hbm.at[s1, pl.ds(0, 8), pl.ds(wid, 1)],
        )

    fire(0, 0)

    def body(p, carry):
        s1 = 2 * p
        fire(s1 + 1, 1)
        proc(s1, 0)

        @pl.when(s1 + 2 < _S1)
        def _():
            fire(s1 + 2, 0)

        proc(s1 + 1, 1)
        return carry

    lax.fori_loop(0, _S1 // 2, body, 0)


def kernel(sentence, table):
    table_p = jnp.pad(table, ((0, 0), (0, 128 - DIM)))
    out5 = _emb_lookup(sentence.T.astype(jnp.int32), table_p)
    return out5.transpose(2, 4, 0, 1, 3).reshape(_S0, _S1, DIM)


# speedup vs baseline: 1.0909x; 1.0909x over previous
import functools

import jax
import jax.numpy as jnp
from jax import lax
from jax.experimental import pallas as pl
from jax.experimental.pallas import tpu as pltpu
from jax.experimental.pallas import tpu_sc as plsc

DIM = 64
_S0, _S1 = 4096, 200

_info = plsc.get_sparse_core_info()
_NC, _NS = _info.num_cores, _info.num_subcores
_NW = _NC * _NS
_TPW = _S0 // _NW
_GRP = 8
_NG = _S1 // _GRP

_mesh = plsc.VectorSubcoreMesh(core_axis_name="c", subcore_axis_name="s")


@functools.partial(
    pl.kernel,
    mesh=_mesh,
    out_type=jax.ShapeDtypeStruct((_S1, 8, _S0 // 128, 8, 128), jnp.float32),
    scratch_types=[
        pltpu.VMEM((2, _GRP, 128), jnp.int32),
        pltpu.VMEM((4, 128, 128), jnp.float32),
        pltpu.VMEM((4, 8, 1, 8, 128), jnp.float32),
        pltpu.SemaphoreType.DMA,
        pltpu.SemaphoreType.DMA,
        pltpu.SemaphoreType.DMA,
        pltpu.SemaphoreType.DMA,
        pltpu.SemaphoreType.DMA,
        pltpu.SemaphoreType.DMA,
        pltpu.SemaphoreType.DMA,
        pltpu.SemaphoreType.DMA,
        pltpu.SemaphoreType.DMA,
    ],
    compiler_params=pltpu.CompilerParams(needs_layout_passes=False),
)
def _emb_lookup(
    sent_hbm, table_hbm, out_hbm, idx_v, rows_v, trans_v,
    isem, gsem0, gsem1, gsem2, gsem3, wsem0, wsem1, wsem2, wsem3,
):
    wid = lax.axis_index("s") * _NC + lax.axis_index("c")
    col0 = pl.multiple_of(wid * _TPW, _TPW)
    gsems = (gsem0, gsem1, gsem2, gsem3)
    wsems = (wsem0, wsem1, wsem2, wsem3)
    lanes = [jnp.arange(16, dtype=jnp.int32) + 16 * j for j in range(8)]

    def fire_idx(g):
        r0 = pl.multiple_of(g * _GRP, _GRP)
        pltpu.async_copy(
            sent_hbm.at[pl.ds(r0, _GRP), pl.ds(col0, _TPW)],
            idx_v.at[g % 2], isem,
        )

    def wait_idx(g):
        r0 = pl.multiple_of(g * _GRP, _GRP)
        pltpu.make_async_copy(
            sent_hbm.at[pl.ds(r0, _GRP), pl.ds(col0, _TPW)],
            idx_v.at[g % 2], isem,
        ).wait()

    def fire_gather(s1, slot):
        pltpu.async_copy(
            table_hbm.at[idx_v.at[(s1 // _GRP) % 2, s1 % _GRP]],
            rows_v.at[slot], gsems[slot],
        )

    def drain_gather(s1, slot):
        pltpu.make_async_copy(
            table_hbm.at[idx_v.at[(s1 // _GRP) % 2, s1 % _GRP]],
            rows_v.at[slot], gsems[slot],
        ).wait()

    def out_ref(s1):
        return out_hbm.at[s1, pl.ds(0, 8), pl.ds(wid, 1)]

    def proc(s1, slot, tb):
        drain_gather(s1, slot)

        def trow(fd, carry):
            for fm in range(8):
                col = jnp.full((16,), fd * 8 + fm, dtype=jnp.int32)
                for j in range(8):
                    v = plsc.load_gather(rows_v.at[slot], [lanes[j], col])
                    trans_v[tb, fd, 0, fm, pl.ds(16 * j, 16)] = v
            return carry

        lax.fori_loop(0, 8, trow, 0)
        pltpu.async_copy(trans_v.at[tb], out_ref(s1), wsems[tb])

    def wait_write(s1, tb):
        pltpu.make_async_copy(trans_v.at[tb], out_ref(s1), wsems[tb]).wait()

    fire_idx(0)
    wait_idx(0)
    fire_idx(1)
    for k in range(4):
        fire_gather(k, k)

    def body(p, carry):
        s1b = 4 * p
        is_odd = (p % 2) == 1
        gnew = (s1b + 4) // _GRP

        @pl.when(jnp.logical_and(is_odd, gnew < _NG))
        def _():
            wait_idx(gnew)

        for k in range(4):
            s1 = s1b + k

            @pl.when(s1 >= 4)
            def _():
                wait_write(s1 - 4, k)

            proc(s1, k, k)

            @pl.when(s1 + 4 < _S1)
            def _():
                fire_gather(s1 + 4, k)

        @pl.when(jnp.logical_and(is_odd, gnew + 1 < _NG))
        def _():
            fire_idx(gnew + 1)

        return carry

    lax.fori_loop(0, _S1 // 4, body, 0)
    for k in range(4):
        wait_write(_S1 - 4 + k, k)


def kernel(sentence, table):
    table_p = jnp.pad(table, ((0, 0), (0, 128 - DIM)))
    out5 = _emb_lookup(sentence.T.astype(jnp.int32), table_p)
    return out5.transpose(2, 4, 0, 1, 3).reshape(_S0, _S1, DIM)


# speedup vs baseline: 1.5906x; 1.4581x over previous
import functools

import jax
import jax.numpy as jnp
from jax import lax
from jax.experimental import pallas as pl
from jax.experimental.pallas import tpu as pltpu
from jax.experimental.pallas import tpu_sc as plsc

DIM = 64
_S0, _S1 = 4096, 200

_info = plsc.get_sparse_core_info()
_NC, _NS = _info.num_cores, _info.num_subcores
_NW = _NC * _NS
_TPW = _S0 // _NW
_GRP = 8
_NG = _S1 // _GRP

_mesh = plsc.VectorSubcoreMesh(core_axis_name="c", subcore_axis_name="s")


@functools.partial(
    pl.kernel,
    mesh=_mesh,
    out_type=jax.ShapeDtypeStruct((_S1, 8, _S0 // 128, 8, 128), jnp.float32),
    scratch_types=[
        pltpu.VMEM((2, _GRP, 128), jnp.int32),
        pltpu.VMEM((4, 128, 128), jnp.float32),
        pltpu.VMEM((8, 2, 1, 1, 8, 128), jnp.float32),
        pltpu.SemaphoreType.DMA,
        pltpu.SemaphoreType.DMA,
        pltpu.SemaphoreType.DMA,
        pltpu.SemaphoreType.DMA,
        pltpu.SemaphoreType.DMA,
        pltpu.SemaphoreType.DMA,
        pltpu.SemaphoreType.DMA,
    ],
    compiler_params=pltpu.CompilerParams(needs_layout_passes=False),
)
def _emb_lookup(
    sent_hbm, table_hbm, out_hbm, idx_v, rows_v, trans_v,
    isem, gsem0, gsem1, gsem2, gsem3, wsem0, wsem1,
):
    wid = lax.axis_index("s") * _NC + lax.axis_index("c")
    col0 = pl.multiple_of(wid * _TPW, _TPW)
    gsems = (gsem0, gsem1, gsem2, gsem3)
    wsems = (wsem0, wsem1)
    iota = jnp.arange(16, dtype=jnp.int32)
    t_j = [iota + 16 * j for j in range(8)]
    fm_d = [(iota + d) % 8 for d in range(8)]
    zero16 = iota * 0

    def fire_idx(g):
        r0 = pl.multiple_of(g * _GRP, _GRP)
        pltpu.async_copy(
            sent_hbm.at[pl.ds(r0, _GRP), pl.ds(col0, _TPW)],
            idx_v.at[g % 2], isem,
        )

    def wait_idx(g):
        r0 = pl.multiple_of(g * _GRP, _GRP)
        pltpu.make_async_copy(
            sent_hbm.at[pl.ds(r0, _GRP), pl.ds(col0, _TPW)],
            idx_v.at[g % 2], isem,
        ).wait()

    def fire_gather(s1, slot):
        pltpu.async_copy(
            table_hbm.at[idx_v.at[(s1 // _GRP) % 2, s1 % _GRP]],
            rows_v.at[slot], gsems[slot],
        )

    def drain_gather(s1, slot):
        pltpu.make_async_copy(
            table_hbm.at[idx_v.at[(s1 // _GRP) % 2, s1 % _GRP]],
            rows_v.at[slot], gsems[slot],
        ).wait()

    def out_ref(s1, fd):
        return out_hbm.at[s1, pl.ds(fd, 1), pl.ds(wid, 1)]

    def wait_writes(s1, par):
        for _ in range(8):
            pltpu.make_async_copy(
                trans_v.at[0, par], out_ref(s1, 0), wsems[par]
            ).wait()

    def proc(s1, slot, par):
        drain_gather(s1, slot)

        def trow(fd, carry):
            dst = trans_v.at[fd, par]
            for d in range(8):
                col = fm_d[d] + fd * 8
                for j in range(8):
                    v = plsc.load_gather(rows_v.at[slot], [t_j[j], col])
                    plsc.store_scatter(dst, [zero16, zero16, fm_d[d], t_j[j]], v)
            pltpu.async_copy(dst, out_ref(s1, fd), wsems[par])
            return carry

        lax.fori_loop(0, 8, trow, 0)

    fire_idx(0)
    wait_idx(0)
    fire_idx(1)
    for k in range(4):
        fire_gather(k, k)

    def body(p, carry):
        s1b = 4 * p
        is_odd = (p % 2) == 1
        gnew = (s1b + 4) // _GRP

        @pl.when(jnp.logical_and(is_odd, gnew < _NG))
        def _():
            wait_idx(gnew)

        for k in range(4):
            s1 = s1b + k

            @pl.when(s1 >= 2)
            def _():
                wait_writes(s1 - 2, k % 2)

            proc(s1, k, k % 2)

            @pl.when(s1 + 4 < _S1)
            def _():
                fire_gather(s1 + 4, k)

        @pl.when(jnp.logical_and(is_odd, gnew + 1 < _NG))
        def _():
            fire_idx(gnew + 1)

        return carry

    lax.fori_loop(0, _S1 // 4, body, 0)
    wait_writes(_S1 - 2, 0)
    wait_writes(_S1 - 1, 1)


def kernel(sentence, table):
    table_p = jnp.pad(table, ((0, 0), (0, 128 - DIM)))
    out5 = _emb_lookup(sentence.T.astype(jnp.int32), table_p)
    return out5.transpose(2, 4, 0, 1, 3).reshape(_S0, _S1, DIM)


# speedup vs baseline: 1.8186x; 1.1434x over previous
import functools

import jax
import jax.numpy as jnp
from jax import lax
from jax.experimental import pallas as pl
from jax.experimental.pallas import tpu as pltpu
from jax.experimental.pallas import tpu_sc as plsc

DIM = 64
_S0, _S1 = 4096, 200

_info = plsc.get_sparse_core_info()
_NC, _NS = _info.num_cores, _info.num_subcores
_NW = _NC * _NS
_TPW = _S0 // _NW
_GRP = 8
_NG = _S1 // _GRP

_mesh = plsc.VectorSubcoreMesh(core_axis_name="c", subcore_axis_name="s")


@functools.partial(
    pl.kernel,
    mesh=_mesh,
    out_type=jax.ShapeDtypeStruct((_S1, 8, _S0 // 128, 8, 128), jnp.float32),
    scratch_types=[
        pltpu.VMEM((2, _GRP, 2, 64), jnp.int32),
        pltpu.VMEM((8, 64, 128), jnp.float32),
        pltpu.VMEM((8, 2, 1, 1, 8, 128), jnp.float32),
        pltpu.SemaphoreType.DMA,
        pltpu.SemaphoreType.DMA,
        pltpu.SemaphoreType.DMA,
        pltpu.SemaphoreType.DMA,
        pltpu.SemaphoreType.DMA,
        pltpu.SemaphoreType.DMA,
        pltpu.SemaphoreType.DMA,
        pltpu.SemaphoreType.DMA,
        pltpu.SemaphoreType.DMA,
        pltpu.SemaphoreType.DMA,
        pltpu.SemaphoreType.DMA,
    ],
    compiler_params=pltpu.CompilerParams(needs_layout_passes=False),
)
def _emb_lookup(
    sent_hbm, table_hbm, out_hbm, idx_v, rows_v, trans_v,
    isem, gsem0, gsem1, gsem2, gsem3, gsem4, gsem5, gsem6, gsem7,
    wsem0, wsem1,
):
    wid = lax.axis_index("s") * _NC + lax.axis_index("c")
    col0 = pl.multiple_of(wid * _TPW, _TPW)
    gsems = (gsem0, gsem1, gsem2, gsem3, gsem4, gsem5, gsem6, gsem7)
    wsems = (wsem0, wsem1)
    iota = jnp.arange(16, dtype=jnp.int32)
    t_j = [iota + 16 * j for j in range(8)]
    fm_d = [(iota + d) % 8 for d in range(8)]
    zero16 = iota * 0

    def fire_idx(g):
        r0 = pl.multiple_of(g * _GRP, _GRP)
        pltpu.async_copy(
            sent_hbm.at[pl.ds(r0, _GRP), pl.ds(2 * wid, 2)],
            idx_v.at[g % 2], isem,
        )

    def wait_idx(g):
        r0 = pl.multiple_of(g * _GRP, _GRP)
        pltpu.make_async_copy(
            sent_hbm.at[pl.ds(r0, _GRP), pl.ds(2 * wid, 2)],
            idx_v.at[g % 2], isem,
        ).wait()

    def fire_gather(s1, h, slot):
        pltpu.async_copy(
            table_hbm.at[idx_v.at[(s1 // _GRP) % 2, s1 % _GRP, h]],
            rows_v.at[slot], gsems[slot],
        )

    def drain_gather(s1, h, slot):
        pltpu.make_async_copy(
            table_hbm.at[idx_v.at[(s1 // _GRP) % 2, s1 % _GRP, h]],
            rows_v.at[slot], gsems[slot],
        ).wait()

    def out_ref(s1, fd):
        return out_hbm.at[s1, pl.ds(fd, 1), pl.ds(wid, 1)]

    def wait_writes(s1, par):
        for _ in range(8):
            pltpu.make_async_copy(
                trans_v.at[0, par], out_ref(s1, 0), wsems[par]
            ).wait()

    def proc(s1, slotA, slotB, par):
        drain_gather(s1, 0, slotA)
        drain_gather(s1, 1, slotB)

        def trow(fd, carry):
            dst = trans_v.at[fd, par]
            for d in range(8):
                col = fm_d[d] + fd * 8
                for j in range(8):
                    src_slot = slotA if j < 4 else slotB
                    v = plsc.load_gather(
                        rows_v.at[src_slot], [t_j[j % 4], col]
                    )
                    plsc.store_scatter(dst, [zero16, zero16, fm_d[d], t_j[j]], v)
            pltpu.async_copy(dst, out_ref(s1, fd), wsems[par])
            return carry

        lax.fori_loop(0, 8, trow, 0)

    fire_idx(0)
    wait_idx(0)
    fire_idx(1)
    for k in range(4):
        fire_gather(k, 0, 2 * k)
        fire_gather(k, 1, 2 * k + 1)

    def body(p, carry):
        s1b = 4 * p
        is_odd = (p % 2) == 1
        gnew = (s1b + 4) // _GRP

        @pl.when(jnp.logical_and(is_odd, gnew < _NG))
        def _():
            wait_idx(gnew)

        for k in range(4):
            s1 = s1b + k

            @pl.when(s1 >= 2)
            def _():
                wait_writes(s1 - 2, k % 2)

            proc(s1, 2 * k, 2 * k + 1, k % 2)

            @pl.when(s1 + 4 < _S1)
            def _():
                fire_gather(s1 + 4, 0, 2 * k)
                fire_gather(s1 + 4, 1, 2 * k + 1)

        @pl.when(jnp.logical_and(is_odd, gnew + 1 < _NG))
        def _():
            fire_idx(gnew + 1)

        return carry

    lax.fori_loop(0, _S1 // 4, body, 0)
    wait_writes(_S1 - 2, 0)
    wait_writes(_S1 - 1, 1)


def kernel(sentence, table):
    table_p = jnp.pad(table, ((0, 0), (0, 128 - DIM)))
    sent3 = sentence.T.astype(jnp.int32).reshape(_S1, _S0 // 64, 64)
    out5 = _emb_lookup(sent3, table_p)
    return out5.transpose(2, 4, 0, 1, 3).reshape(_S0, _S1, DIM)


# speedup vs baseline: 1.8891x; 1.0387x over previous
import functools

import jax
import jax.numpy as jnp
from jax import lax
from jax.experimental import pallas as pl
from jax.experimental.pallas import tpu as pltpu
from jax.experimental.pallas import tpu_sc as plsc

DIM = 64
_S0, _S1 = 4096, 200

_info = plsc.get_sparse_core_info()
_NC, _NS = _info.num_cores, _info.num_subcores
_NW = _NC * _NS
_TPW = _S0 // _NW
_GRP = 8
_NG = _S1 // _GRP

_mesh = plsc.VectorSubcoreMesh(core_axis_name="c", subcore_axis_name="s")


@functools.partial(
    pl.kernel,
    mesh=_mesh,
    out_type=jax.ShapeDtypeStruct((_S1, 8, _S0 // 128, 8, 128), jnp.float32),
    scratch_types=[
        pltpu.VMEM((2, _GRP, 4, 32), jnp.int32),
        pltpu.VMEM((16, 32, 128), jnp.float32),
        pltpu.VMEM((8, 2, 1, 1, 8, 128), jnp.float32),
        pltpu.SemaphoreType.DMA,
        pltpu.SemaphoreType.DMA,
        pltpu.SemaphoreType.DMA,
        pltpu.SemaphoreType.DMA,
        pltpu.SemaphoreType.DMA,
        pltpu.SemaphoreType.DMA,
        pltpu.SemaphoreType.DMA,
        pltpu.SemaphoreType.DMA,
        pltpu.SemaphoreType.DMA,
        pltpu.SemaphoreType.DMA,
        pltpu.SemaphoreType.DMA,
        pltpu.SemaphoreType.DMA,
        pltpu.SemaphoreType.DMA,
        pltpu.SemaphoreType.DMA,
        pltpu.SemaphoreType.DMA,
        pltpu.SemaphoreType.DMA,
        pltpu.SemaphoreType.DMA,
        pltpu.SemaphoreType.DMA,
        pltpu.SemaphoreType.DMA,
    ],
    compiler_params=pltpu.CompilerParams(needs_layout_passes=False),
)
def _emb_lookup(
    sent_hbm, table_hbm, out_hbm, idx_v, rows_v, trans_v,
    isem, gsem0, gsem1, gsem2, gsem3, gsem4, gsem5, gsem6, gsem7,
    gsem8, gsem9, gsem10, gsem11, gsem12, gsem13, gsem14, gsem15,
    wsem0, wsem1,
):
    wid = lax.axis_index("s") * _NC + lax.axis_index("c")
    col0 = pl.multiple_of(wid * _TPW, _TPW)
    gsems = (gsem0, gsem1, gsem2, gsem3, gsem4, gsem5, gsem6, gsem7,
             gsem8, gsem9, gsem10, gsem11, gsem12, gsem13, gsem14, gsem15)
    wsems = (wsem0, wsem1)
    iota = jnp.arange(16, dtype=jnp.int32)
    t_j = [iota + 16 * j for j in range(8)]
    fm_d = [(iota + d) % 8 for d in range(8)]
    zero16 = iota * 0

    def fire_idx(g):
        r0 = pl.multiple_of(g * _GRP, _GRP)
        pltpu.async_copy(
            sent_hbm.at[pl.ds(r0, _GRP), pl.ds(4 * wid, 4)],
            idx_v.at[g % 2], isem,
        )

    def wait_idx(g):
        r0 = pl.multiple_of(g * _GRP, _GRP)
        pltpu.make_async_copy(
            sent_hbm.at[pl.ds(r0, _GRP), pl.ds(4 * wid, 4)],
            idx_v.at[g % 2], isem,
        ).wait()

    def fire_gather(s1, h, slot):
        pltpu.async_copy(
            table_hbm.at[idx_v.at[(s1 // _GRP) % 2, s1 % _GRP, h]],
            rows_v.at[slot], gsems[slot],
        )

    def drain_gather(s1, h, slot):
        pltpu.make_async_copy(
            table_hbm.at[idx_v.at[(s1 // _GRP) % 2, s1 % _GRP, h]],
            rows_v.at[slot], gsems[slot],
        ).wait()

    def out_ref(s1, fd):
        return out_hbm.at[s1, pl.ds(fd, 1), pl.ds(wid, 1)]

    def wait_writes(s1, par):
        for _ in range(8):
            pltpu.make_async_copy(
                trans_v.at[0, par], out_ref(s1, 0), wsems[par]
            ).wait()

    def proc(s1, slots, par):
        for q in range(4):
            drain_gather(s1, q, slots[q])

        def trow(fd, carry):
            dst = trans_v.at[fd, par]
            for d in range(8):
                col = fm_d[d] + fd * 8
                for j in range(8):
                    v = plsc.load_gather(
                        rows_v.at[slots[j // 2]], [t_j[j % 2], col]
                    )
                    plsc.store_scatter(dst, [zero16, zero16, fm_d[d], t_j[j]], v)
            pltpu.async_copy(dst, out_ref(s1, fd), wsems[par])
            return carry

        lax.fori_loop(0, 8, trow, 0)

    fire_idx(0)
    wait_idx(0)
    fire_idx(1)
    for k in range(4):
        for q in range(4):
            fire_gather(k, q, 4 * k + q)

    def body(p, carry):
        s1b = 4 * p
        is_odd = (p % 2) == 1
        gnew = (s1b + 4) // _GRP

        @pl.when(jnp.logical_and(is_odd, gnew < _NG))
        def _():
            wait_idx(gnew)

        for k in range(4):
            s1 = s1b + k

            @pl.when(s1 >= 2)
            def _():
                wait_writes(s1 - 2, k % 2)

            proc(s1, [4 * k, 4 * k + 1, 4 * k + 2, 4 * k + 3], k % 2)

            @pl.when(s1 + 4 < _S1)
            def _():
                for q in range(4):
                    fire_gather(s1 + 4, q, 4 * k + q)

        @pl.when(jnp.logical_and(is_odd, gnew + 1 < _NG))
        def _():
            fire_idx(gnew + 1)

        return carry

    lax.fori_loop(0, _S1 // 4, body, 0)
    wait_writes(_S1 - 2, 0)
    wait_writes(_S1 - 1, 1)


def kernel(sentence, table):
    table_p = jnp.pad(table, ((0, 0), (0, 128 - DIM)))
    sent3 = sentence.T.astype(jnp.int32).reshape(_S1, _S0 // 32, 32)
    out5 = _emb_lookup(sent3, table_p)
    return out5.transpose(2, 4, 0, 1, 3).reshape(_S0, _S1, DIM)
